# baseline (speedup 1.0000x reference)
import functools

import jax
import jax.numpy as jnp
from jax import lax
from jax.experimental import pallas as pl
from jax.experimental.pallas import tpu as pltpu
from jax.experimental.pallas import tpu_sc as plsc

_N = 100000
_D = 256
_B = 64
_H = 16
_L = 16
_NC = 2
_NS = 16
_NW = _NC * _NS
_SEG_PER_W = _B // _NW
_CH = 6272
_NP = _NS * _CH
_XB = 64
_NJ = _D // _L


def _sc_attn_body(x_hbm, ids_hbm, w1_hbm, b1_hbm, w2t_hbm, b2_hbm,
                  attn_hbm, ids_v, xb0_v, xb1_v, xb2_v, xb3_v, xb4_v, xb5_v,
                  tab_v, mtab_v, w1_v, w2t_v, b1_v, b2_v, attnb_v, shtab_v,
                  sem0, sem1, sem2, sem3, sem4, sem5):
    c = lax.axis_index("c")
    s = lax.axis_index("s")
    wid = s * _NC + c
    iota = lax.broadcasted_iota(jnp.int32, (_L,), 0)

    nblk = _CH // _L
    pltpu.sync_copy(ids_hbm.at[pl.ds(s * _CH, _CH)], ids_v)

    def lb_body(b, tab):
        pb = jnp.zeros((), jnp.int32)
        bit = 512
        while bit:
            npb = pb + bit
            vec = ids_v[pl.ds((jnp.minimum(npb, nblk) - 1) * _L, _L)]
            first = jnp.sum(jnp.where(iota == 0, vec, 0))
            take = jnp.logical_and(npb <= nblk, first < b)
            pb = jnp.where(take, npb, pb)
            bit //= 2
        last = ids_v[pl.ds(jnp.maximum(pb - 1, 0) * _L, _L)]
        cnt = jnp.sum((last < b).astype(jnp.int32))
        pos = jnp.where(pb == 0, 0, (pb - 1) * _L + cnt)
        return tuple(jnp.where(j * _L + iota == b, pos, tab[j])
                     for j in range(5))

    tab = lax.fori_loop(
        0, _B + 1, lb_body,
        tuple(jnp.zeros((_L,), jnp.int32) for _ in range(5)))
    for j in range(5):
        tab_v[pl.ds(j * _L, _L)] = tab[j]

    pltpu.sync_copy(tab_v, shtab_v.at[s])
    plsc.subcore_barrier()
    pltpu.sync_copy(shtab_v, mtab_v)

    tot = [jnp.zeros((_L,), jnp.int32) for _ in range(5)]
    for t in range(_NS):
        for j in range(5):
            tot[j] = tot[j] + mtab_v[t, pl.ds(j * _L, _L)]

    def boundary(b):
        m = jnp.zeros((), jnp.int32)
        for j in range(5):
            m = m + jnp.sum(jnp.where(j * _L + iota == b, tot[j], 0))
        return m

    b0 = _SEG_PER_W * wid
    lo = boundary(b0)
    mid = boundary(b0 + 1)
    hi = boundary(b0 + 2)

    bufs = (xb0_v, xb1_v, xb2_v, xb3_v, xb4_v, xb5_v)
    sems = (sem0, sem1, sem2, sem3, sem4, sem5)

    def start(buf, sem, kb):
        bs = jnp.minimum(kb * _XB, _N - _XB)
        pltpu.make_async_copy(x_hbm.at[pl.ds(bs, _XB)], buf, sem).start()

    def wait(buf, sem):
        pltpu.make_async_copy(x_hbm.at[pl.ds(0, _XB)], buf, sem).wait()

    def seg_sum(rlo, rhi):
        def proc(buf, kb, acc):
            bs = jnp.minimum(kb * _XB, _N - _XB)
            r0 = jnp.maximum(rlo, kb * _XB)
            r1 = jnp.minimum(rhi, (kb + 1) * _XB)

            def row_body(r, a):
                rl = r - bs
                return tuple(a[j] + buf[rl, pl.ds(j * _L, _L)]
                             for j in range(_NJ))

            return lax.fori_loop(r0, r1, row_body, acc)

        kb_lo = rlo // _XB
        kb_hi = (rhi + _XB - 1) // _XB

        nring = 6
        for u in range(nring):
            pl.when(kb_lo + u < kb_hi)(
                lambda u=u: start(bufs[u], sems[u], kb_lo + u))

        def ring_body(q, acc):
            kb0 = kb_lo + nring * q
            for u in range(nring):
                kb = kb0 + u
                pl.when(kb < kb_hi)(lambda u=u: wait(bufs[u], sems[u]))
                acc = proc(bufs[u], kb, acc)
                pl.when(kb + nring < kb_hi)(
                    lambda u=u, kb=kb: start(bufs[u], sems[u], kb + nring))
            return acc

        acc0 = tuple(jnp.zeros((_L,), jnp.float32) for _ in range(_NJ))
        return lax.fori_loop(0, (kb_hi - kb_lo + nring - 1) // 6,
                             ring_body, acc0)

    pltpu.sync_copy(w1_hbm, w1_v)
    pltpu.sync_copy(w2t_hbm, w2t_v)
    pltpu.sync_copy(b1_hbm, b1_v)
    pltpu.sync_copy(b2_hbm, b2_v)
    b1vec = b1_v[...]

    def mlp_row(acc, cnt, out_slot):
        cntf = jnp.maximum(cnt.astype(jnp.float32), 1.0)
        inv = 1.0 / jnp.broadcast_to(cntf, (_L,))
        avg = [a * inv for a in acc]
        hs = []
        for i in range(_H):
            d = jnp.zeros((_L,), jnp.float32)
            for j in range(_NJ):
                d = d + avg[j] * w1_v[i, pl.ds(j * _L, _L)]
            b1_i = jnp.sum(jnp.where(iota == i, b1vec, 0.0))
            hs.append(jnp.maximum(jnp.sum(d) + b1_i, 0.0))
        for j in range(_NJ):
            z = b2_v[pl.ds(j * _L, _L)]
            for i in range(_H):
                z = z + hs[i] * w2t_v[i, pl.ds(j * _L, _L)]
            attnb_v[out_slot, pl.ds(j * _L, _L)] = 1.0 / (1.0 + jnp.exp(-z))

    mlp_row(seg_sum(lo, mid), mid - lo, 0)
    mlp_row(seg_sum(mid, hi), hi - mid, 1)
    pltpu.sync_copy(attnb_v, attn_hbm.at[pl.ds(_SEG_PER_W * wid, _SEG_PER_W)])


def _sc_attn(x_input, idsp, W1, b1, W2T, b2):
    mesh = plsc.VectorSubcoreMesh(core_axis_name="c", subcore_axis_name="s")
    fn = pl.kernel(
        _sc_attn_body, mesh=mesh,
        out_type=jax.ShapeDtypeStruct((_B, _D), jnp.float32),
        scratch_types=[
            pltpu.VMEM((_CH,), jnp.int32),
            pltpu.VMEM((_XB, _D), jnp.float32),
            pltpu.VMEM((_XB, _D), jnp.float32),
            pltpu.VMEM((_XB, _D), jnp.float32),
            pltpu.VMEM((_XB, _D), jnp.float32),
            pltpu.VMEM((_XB, _D), jnp.float32),
            pltpu.VMEM((_XB, _D), jnp.float32),
            pltpu.VMEM((128,), jnp.int32),
            pltpu.VMEM((_NS, 128), jnp.int32),
            pltpu.VMEM((_H, _D), jnp.float32),
            pltpu.VMEM((_H, _D), jnp.float32),
            pltpu.VMEM((_H,), jnp.float32),
            pltpu.VMEM((_D,), jnp.float32),
            pltpu.VMEM((_SEG_PER_W, _D), jnp.float32),
            pltpu.VMEM_SHARED((_NS, 128), jnp.int32),
            pltpu.SemaphoreType.DMA,
            pltpu.SemaphoreType.DMA,
            pltpu.SemaphoreType.DMA,
            pltpu.SemaphoreType.DMA,
            pltpu.SemaphoreType.DMA,
            pltpu.SemaphoreType.DMA,
        ],
        compiler_params=pltpu.CompilerParams(needs_layout_passes=False),
    )
    return fn(x_input, idsp, W1, b1, W2T, b2)


_BLK = 5000
_STEPS = _N // _BLK


def _tc_rescale_body(x_ref, ids_ref, attn_ref, out_ref):
    ids = ids_ref[0, 0, :]
    onehot = (ids[:, None] == lax.broadcasted_iota(
        jnp.int32, (_BLK, _B), 1)).astype(jnp.float32)
    scale = lax.dot_general(
        onehot, attn_ref[...], (((1,), (0,)), ((), ())),
        preferred_element_type=jnp.float32)
    out_ref[...] = x_ref[...] * scale


def _tc_rescale(x_input, ids3, attn):
    return pl.pallas_call(
        _tc_rescale_body,
        grid=(_STEPS,),
        in_specs=[
            pl.BlockSpec((_BLK, _D), lambda i: (i, 0)),
            pl.BlockSpec((1, 1, _BLK), lambda i: (i, 0, 0)),
            pl.BlockSpec((_B, _D), lambda i: (0, 0)),
        ],
        out_specs=pl.BlockSpec((_BLK, _D), lambda i: (i, 0)),
        out_shape=jax.ShapeDtypeStruct((_N, _D), jnp.float32),
        compiler_params=pltpu.CompilerParams(
            dimension_semantics=("arbitrary",)),
    )(x_input, ids3, attn)


def kernel(x_input, W1, b1, W2, b2, batch, batch_num):
    del batch_num
    ids = batch.astype(jnp.int32)
    idsp = jnp.concatenate([ids, jnp.full((_NP - _N,), _B, jnp.int32)])
    attn = _sc_attn(x_input, idsp, W1, b1, W2.T, b2)
    ids3 = ids.reshape(_STEPS, 1, _BLK)
    return _tc_rescale(x_input, ids3, attn)


# speedup vs baseline: 1.0116x; 1.0116x over previous
import functools

import jax
import jax.numpy as jnp
from jax import lax
from jax.experimental import pallas as pl
from jax.experimental.pallas import tpu as pltpu
from jax.experimental.pallas import tpu_sc as plsc

_N = 100000
_D = 256
_B = 64
_H = 16
_L = 16
_NC = 2
_NS = 16
_NW = _NC * _NS
_SEG_PER_W = _B // _NW
_CH = 6272
_NP = _NS * _CH
_XB = 64
_NJ = _D // _L


def _sc_attn_body(x_hbm, ids_hbm, w1_hbm, b1_hbm, w2t_hbm, b2_hbm,
                  attn_hbm, ids_v, xb0_v, xb1_v, xb2_v, xb3_v, tab_v, mtab_v,
                  w1_v, w2t_v, b1_v, b2_v, attnb_v, shtab_v,
                  sem0, sem1, sem2, sem3):
    c = lax.axis_index("c")
    s = lax.axis_index("s")
    wid = s * _NC + c
    iota = lax.broadcasted_iota(jnp.int32, (_L,), 0)

    nblk = _CH // _L
    pltpu.sync_copy(ids_hbm.at[pl.ds(s * _CH, _CH)], ids_v)

    def lb_body(b, tab):
        pb = jnp.zeros((), jnp.int32)
        bit = 512
        while bit:
            npb = pb + bit
            vec = ids_v[pl.ds((jnp.minimum(npb, nblk) - 1) * _L, _L)]
            first = jnp.sum(jnp.where(iota == 0, vec, 0))
            take = jnp.logical_and(npb <= nblk, first < b)
            pb = jnp.where(take, npb, pb)
            bit //= 2
        last = ids_v[pl.ds(jnp.maximum(pb - 1, 0) * _L, _L)]
        cnt = jnp.sum((last < b).astype(jnp.int32))
        pos = jnp.where(pb == 0, 0, (pb - 1) * _L + cnt)
        return tuple(jnp.where(j * _L + iota == b, pos, tab[j])
                     for j in range(5))

    tab = lax.fori_loop(
        0, _B + 1, lb_body,
        tuple(jnp.zeros((_L,), jnp.int32) for _ in range(5)))
    for j in range(5):
        tab_v[pl.ds(j * _L, _L)] = tab[j]

    pltpu.sync_copy(tab_v, shtab_v.at[s])
    plsc.subcore_barrier()
    pltpu.sync_copy(shtab_v, mtab_v)

    tot = [jnp.zeros((_L,), jnp.int32) for _ in range(5)]
    for t in range(_NS):
        for j in range(5):
            tot[j] = tot[j] + mtab_v[t, pl.ds(j * _L, _L)]

    def boundary(b):
        m = jnp.zeros((), jnp.int32)
        for j in range(5):
            m = m + jnp.sum(jnp.where(j * _L + iota == b, tot[j], 0))
        return m

    b0 = _SEG_PER_W * wid
    lo = boundary(b0)
    mid = boundary(b0 + 1)
    hi = boundary(b0 + 2)

    bufs = (xb0_v, xb1_v, xb2_v, xb3_v)
    sems = (sem0, sem1, sem2, sem3)

    def start(buf, sem, kb):
        bs = jnp.minimum(kb * _XB, _N - _XB)
        pltpu.make_async_copy(x_hbm.at[pl.ds(bs, _XB)], buf, sem).start()

    def wait(buf, sem):
        pltpu.make_async_copy(x_hbm.at[pl.ds(0, _XB)], buf, sem).wait()

    def seg_sum(rlo, rhi):
        def proc(buf, kb, acc):
            bs = jnp.minimum(kb * _XB, _N - _XB)
            r0 = jnp.maximum(rlo, kb * _XB)
            r1 = jnp.minimum(rhi, (kb + 1) * _XB)

            def row_body(r, a):
                rl = r - bs
                return tuple(a[j] + buf[rl, pl.ds(j * _L, _L)]
                             for j in range(_NJ))

            return lax.fori_loop(r0, r1, row_body, acc)

        kb_lo = rlo // _XB
        kb_hi = (rhi + _XB - 1) // _XB

        nring = 4
        for u in range(nring):
            pl.when(kb_lo + u < kb_hi)(
                lambda u=u: start(bufs[u], sems[u], kb_lo + u))

        def ring_body(q, acc):
            kb0 = kb_lo + nring * q
            for u in range(nring):
                kb = kb0 + u
                pl.when(kb < kb_hi)(lambda u=u: wait(bufs[u], sems[u]))
                acc = proc(bufs[u], kb, acc)
                pl.when(kb + nring < kb_hi)(
                    lambda u=u, kb=kb: start(bufs[u], sems[u], kb + nring))
            return acc

        acc0 = tuple(jnp.zeros((_L,), jnp.float32) for _ in range(_NJ))
        return lax.fori_loop(0, (kb_hi - kb_lo + nring - 1) // 4,
                             ring_body, acc0)

    pltpu.sync_copy(w1_hbm, w1_v)
    pltpu.sync_copy(w2t_hbm, w2t_v)
    pltpu.sync_copy(b1_hbm, b1_v)
    pltpu.sync_copy(b2_hbm, b2_v)
    b1vec = b1_v[...]

    def mlp_row(acc, cnt, out_slot):
        cntf = jnp.maximum(cnt.astype(jnp.float32), 1.0)
        inv = 1.0 / jnp.broadcast_to(cntf, (_L,))
        avg = [a * inv for a in acc]
        hs = []
        for i in range(_H):
            d = jnp.zeros((_L,), jnp.float32)
            for j in range(_NJ):
                d = d + avg[j] * w1_v[i, pl.ds(j * _L, _L)]
            b1_i = jnp.sum(jnp.where(iota == i, b1vec, 0.0))
            hs.append(jnp.maximum(jnp.sum(d) + b1_i, 0.0))
        for j in range(_NJ):
            z = b2_v[pl.ds(j * _L, _L)]
            for i in range(_H):
                z = z + hs[i] * w2t_v[i, pl.ds(j * _L, _L)]
            attnb_v[out_slot, pl.ds(j * _L, _L)] = 1.0 / (1.0 + jnp.exp(-z))

    mlp_row(seg_sum(lo, mid), mid - lo, 0)
    mlp_row(seg_sum(mid, hi), hi - mid, 1)
    pltpu.sync_copy(attnb_v, attn_hbm.at[pl.ds(_SEG_PER_W * wid, _SEG_PER_W)])


def _sc_attn(x_input, idsp, W1, b1, W2T, b2):
    mesh = plsc.VectorSubcoreMesh(core_axis_name="c", subcore_axis_name="s")
    fn = pl.kernel(
        _sc_attn_body, mesh=mesh,
        out_type=jax.ShapeDtypeStruct((_B, _D), jnp.float32),
        scratch_types=[
            pltpu.VMEM((_CH,), jnp.int32),
            pltpu.VMEM((_XB, _D), jnp.float32),
            pltpu.VMEM((_XB, _D), jnp.float32),
            pltpu.VMEM((_XB, _D), jnp.float32),
            pltpu.VMEM((_XB, _D), jnp.float32),
            pltpu.VMEM((128,), jnp.int32),
            pltpu.VMEM((_NS, 128), jnp.int32),
            pltpu.VMEM((_H, _D), jnp.float32),
            pltpu.VMEM((_H, _D), jnp.float32),
            pltpu.VMEM((_H,), jnp.float32),
            pltpu.VMEM((_D,), jnp.float32),
            pltpu.VMEM((_SEG_PER_W, _D), jnp.float32),
            pltpu.VMEM_SHARED((_NS, 128), jnp.int32),
            pltpu.SemaphoreType.DMA,
            pltpu.SemaphoreType.DMA,
            pltpu.SemaphoreType.DMA,
            pltpu.SemaphoreType.DMA,
        ],
        compiler_params=pltpu.CompilerParams(needs_layout_passes=False),
    )
    return fn(x_input, idsp, W1, b1, W2T, b2)


_BLK = 4000
_STEPS = _N // _BLK


def _tc_rescale_body(x_ref, ids_ref, attn_ref, out_ref):
    ids = ids_ref[0, 0, :]
    onehot = (ids[:, None] == lax.broadcasted_iota(
        jnp.int32, (_BLK, _B), 1)).astype(jnp.float32)
    scale = lax.dot_general(
        onehot, attn_ref[...], (((1,), (0,)), ((), ())),
        preferred_element_type=jnp.float32)
    out_ref[...] = x_ref[...] * scale


def _tc_rescale(x_input, ids3, attn):
    return pl.pallas_call(
        _tc_rescale_body,
        grid=(_STEPS,),
        in_specs=[
            pl.BlockSpec((_BLK, _D), lambda i: (i, 0)),
            pl.BlockSpec((1, 1, _BLK), lambda i: (i, 0, 0)),
            pl.BlockSpec((_B, _D), lambda i: (0, 0)),
        ],
        out_specs=pl.BlockSpec((_BLK, _D), lambda i: (i, 0)),
        out_shape=jax.ShapeDtypeStruct((_N, _D), jnp.float32),
        compiler_params=pltpu.CompilerParams(
            dimension_semantics=("arbitrary",)),
    )(x_input, ids3, attn)


def kernel(x_input, W1, b1, W2, b2, batch, batch_num):
    del batch_num
    ids = batch.astype(jnp.int32)
    idsp = jnp.concatenate([ids, jnp.full((_NP - _N,), _B, jnp.int32)])
    attn = _sc_attn(x_input, idsp, W1, b1, W2.T, b2)
    ids3 = ids.reshape(_STEPS, 1, _BLK)
    return _tc_rescale(x_input, ids3, attn)
